# Initial kernel scaffold; baseline (speedup 1.0000x reference)
import jax
import jax.numpy as jnp
from jax.experimental import pallas as pl


def kernel(x, new_centers):
    raise NotImplementedError("write your pallas kernel here")


# speedup vs baseline: 7.8506x; 7.8506x over previous
import jax
import jax.numpy as jnp
from jax import lax
from jax.experimental import pallas as pl

N, K, M = 16384, 256, 64
BLK = 1024
NBLK = N // BLK
NS = 128
NL = 128


def _dist_kernel(xt_ref, c_ref, dt_ref, mind_ref):
    xt = xt_ref[...]
    c = c_ref[...]
    xc = lax.dot_general(c, xt, (((1,), (0,)), ((), ())),
                         preferred_element_type=jnp.float32,
                         precision=lax.Precision.HIGHEST)
    xx = jnp.sum(xt * xt, axis=0)[None, :]
    cc = jnp.sum(c * c, axis=1, keepdims=True)
    d = jnp.sqrt(jnp.maximum(cc + xx - 2.0 * xc, 0.0))
    dt_ref[...] = d
    mind_ref[...] = jnp.min(d, axis=0, keepdims=True)


def _greedy_kernel(dt_ref, idx_ref):
    i0 = lax.broadcasted_iota(jnp.int32, (NS, NL), 0)
    i1 = lax.broadcasted_iota(jnp.int32, (NS, NL), 1)
    iota = i0 * NL + i1
    col_iota = lax.broadcasted_iota(jnp.int32, (1, M), 1)
    penalty = jnp.zeros((NS, NL), jnp.float32)
    idxs = jnp.zeros((1, M), jnp.int32)
    for j in range(M):
        masked = dt_ref[j] + penalty
        mval = jnp.min(masked)
        idx = jnp.min(jnp.where(masked == mval, iota, jnp.int32(N)))
        penalty = jnp.where(iota == idx, jnp.float32(jnp.inf), penalty)
        idxs = jnp.where(col_iota == j, idx, idxs)
    idx_ref[...] = idxs


_dist_call = pl.pallas_call(
    _dist_kernel,
    grid=(NBLK,),
    in_specs=[pl.BlockSpec((K, BLK), lambda i: (0, i)),
              pl.BlockSpec((M, K), lambda i: (0, 0))],
    out_specs=[pl.BlockSpec((M, BLK), lambda i: (0, i)),
               pl.BlockSpec((1, BLK), lambda i: (0, i))],
    out_shape=[jax.ShapeDtypeStruct((M, N), jnp.float32),
               jax.ShapeDtypeStruct((1, N), jnp.float32)],
)

_greedy_call = pl.pallas_call(
    _greedy_kernel,
    out_shape=jax.ShapeDtypeStruct((1, M), jnp.int32),
)


def kernel(x, new_centers):
    xt = x.T
    dt, mind = _dist_call(xt, new_centers)
    idxs = _greedy_call(dt.reshape(M, NS, NL))
    return (idxs.reshape(M).astype(jnp.int64), mind.reshape(N), new_centers)


# speedup vs baseline: 10.8326x; 1.3798x over previous
import jax
import jax.numpy as jnp
from jax import lax
from jax.experimental import pallas as pl
from jax.experimental.pallas import tpu as pltpu

N, K, M = 16384, 256, 64
BLK = 1024
NBLK = N // BLK


def _fused_kernel(x_ref, c_ref, mind_ref, idx_ref, dt_ref):
    i = pl.program_id(0)
    xb = x_ref[...]
    c = c_ref[...]
    xcT = lax.dot_general(c, xb, (((1,), (1,)), ((), ())),
                          preferred_element_type=jnp.float32,
                          precision=lax.Precision.HIGHEST)
    xxT = lax.dot_general(jnp.ones((1, K), jnp.float32), xb * xb,
                          (((1,), (1,)), ((), ())),
                          preferred_element_type=jnp.float32,
                          precision=lax.Precision.HIGHEST)
    cc = jnp.sum(c * c, axis=1, keepdims=True)
    dT = jnp.sqrt(jnp.maximum(cc + xxT - 2.0 * xcT, 0.0))
    dt_ref[i] = dT
    mind_ref[...] = jnp.min(dT, axis=0, keepdims=True)

    @pl.when(i == NBLK - 1)
    def _greedy():
        i0 = lax.broadcasted_iota(jnp.int32, (NBLK, BLK), 0)
        i1 = lax.broadcasted_iota(jnp.int32, (NBLK, BLK), 1)
        iota = i0 * BLK + i1
        col_iota = lax.broadcasted_iota(jnp.int32, (1, M), 1)
        penalty = jnp.zeros((NBLK, BLK), jnp.float32)
        idxs = jnp.zeros((1, M), jnp.int32)
        for j in range(M):
            masked = dt_ref[:, j, :] + penalty
            mval = jnp.min(masked)
            idx = jnp.min(jnp.where(masked == mval, iota, jnp.int32(N)))
            penalty = jnp.where(iota == idx, jnp.float32(jnp.inf), penalty)
            idxs = jnp.where(col_iota == j, idx, idxs)
        idx_ref[...] = idxs


_fused_call = pl.pallas_call(
    _fused_kernel,
    grid=(NBLK,),
    in_specs=[pl.BlockSpec((BLK, K), lambda i: (i, 0)),
              pl.BlockSpec((M, K), lambda i: (0, 0))],
    out_specs=[pl.BlockSpec((1, BLK), lambda i: (0, i)),
               pl.BlockSpec((1, M), lambda i: (0, 0))],
    out_shape=[jax.ShapeDtypeStruct((1, N), jnp.float32),
               jax.ShapeDtypeStruct((1, M), jnp.int32)],
    scratch_shapes=[pltpu.VMEM((NBLK, M, BLK), jnp.float32)],
)


def kernel(x, new_centers):
    mind, idxs = _fused_call(x, new_centers)
    return (idxs.reshape(M).astype(jnp.int64), mind.reshape(N), new_centers)


# speedup vs baseline: 14.5393x; 1.3422x over previous
import jax
import jax.numpy as jnp
from jax import lax
from jax.experimental import pallas as pl
from jax.experimental.pallas import tpu as pltpu

N, K, M = 16384, 256, 64
BLK = 1024
NBLK = N // BLK


def _fused_kernel(x_ref, c_ref, mind_ref, idx_ref, dt_ref):
    i = pl.program_id(0)
    xb = jnp.transpose(x_ref[...])
    c = c_ref[...]
    xcT = lax.dot_general(c, xb, (((1,), (0,)), ((), ())),
                          preferred_element_type=jnp.float32,
                          precision=lax.Precision.HIGHEST)
    xxT = jnp.sum(xb * xb, axis=0, keepdims=True)
    cc = jnp.sum(c * c, axis=1, keepdims=True)
    dT = jnp.sqrt(jnp.maximum(cc + xxT - 2.0 * xcT, 0.0))
    dt_ref[i] = dT
    mind_ref[...] = jnp.min(dT, axis=0, keepdims=True)

    @pl.when(i == NBLK - 1)
    def _greedy():
        i0 = lax.broadcasted_iota(jnp.int32, (NBLK, BLK), 0)
        i1 = lax.broadcasted_iota(jnp.int32, (NBLK, BLK), 1)
        iota = i0 * BLK + i1
        col_iota = lax.broadcasted_iota(jnp.int32, (1, M), 1)
        penalty = jnp.zeros((NBLK, BLK), jnp.float32)
        idxs = jnp.zeros((1, M), jnp.int32)

        def vreg_min_11(a):
            a = jnp.minimum(a[:8], a[8:])
            w = BLK // 2
            while w >= 128:
                a = jnp.minimum(a[:, :w], a[:, w:])
                w //= 2
            return jnp.min(a, keepdims=True)

        for j in range(M):
            masked = dt_ref[:, j, :] + penalty
            mval = vreg_min_11(masked)
            cand = jnp.where(masked == mval, iota, jnp.int32(N))
            ix = vreg_min_11(cand)
            penalty = jnp.where(iota == ix, jnp.float32(jnp.inf), penalty)
            idxs = jnp.where(col_iota == j, ix, idxs)
        idx_ref[...] = idxs


_fused_call = pl.pallas_call(
    _fused_kernel,
    grid=(NBLK,),
    in_specs=[pl.BlockSpec((BLK, K), lambda i: (i, 0)),
              pl.BlockSpec((M, K), lambda i: (0, 0))],
    out_specs=[pl.BlockSpec((1, BLK), lambda i: (0, i)),
               pl.BlockSpec((1, M), lambda i: (0, 0))],
    out_shape=[jax.ShapeDtypeStruct((1, N), jnp.float32),
               jax.ShapeDtypeStruct((1, M), jnp.int32)],
    scratch_shapes=[pltpu.VMEM((NBLK, M, BLK), jnp.float32)],
)


def kernel(x, new_centers):
    mind, idxs = _fused_call(x, new_centers)
    return (idxs.reshape(M).astype(jnp.int64), mind.reshape(N), new_centers)
